# baseline (speedup 1.0000x reference)
import functools

import jax
import jax.numpy as jnp
import numpy as np
from jax import lax
from jax.experimental import pallas as pl
from jax.experimental.pallas import tpu as pltpu
from jax.experimental.pallas import tpu_sc as plsc

VOCAB = 1000000
EMBED = 128
NDENSE = 13
NSPARSE = 26
BATCH = 16384
NFEAT = 32
BOT_ROW = 26
NPAIR = NFEAT * NFEAT

NC = 2
NS = 16
NW = NC * NS

NSLICE = 2
SBATCH = BATCH // NSLICE
B_SLICE = SBATCH * NFEAT
B_PER_W = B_SLICE // NW
CHUNK = 256
N_CHUNKS = B_PER_W // CHUNK
N_PAIRS = N_CHUNKS // 2

BB = 512
GRID = SBATCH // BB


@functools.cache
def _make_sc_gather():
    @functools.partial(
        pl.kernel,
        mesh=plsc.VectorSubcoreMesh(core_axis_name="c", subcore_axis_name="s"),
        out_type=jax.ShapeDtypeStruct((SBATCH * NFEAT, EMBED), jnp.float32),
        scratch_types=[
            pltpu.VMEM((B_PER_W,), jnp.int32),
            pltpu.VMEM((CHUNK, EMBED), jnp.float32),
            pltpu.VMEM((CHUNK, EMBED), jnp.float32),
            pltpu.SemaphoreType.DMA,
            pltpu.SemaphoreType.DMA,
        ],
    )
    def _sc_gather(table_hbm, idx_hbm, out_hbm, idx_v, rows_a, rows_b,
                   sem_a, sem_b):
        wid = lax.axis_index("s") * NC + lax.axis_index("c")
        base = wid * B_PER_W
        pltpu.sync_copy(idx_hbm.at[pl.ds(base, B_PER_W)], idx_v)

        def gather(c, buf, sem):
            off = pl.multiple_of(c * CHUNK, CHUNK)
            return pltpu.async_copy(
                table_hbm.at[idx_v.at[pl.ds(off, CHUNK)]], buf, sem)

        def store(c, buf):
            off = pl.multiple_of(c * CHUNK, CHUNK)
            pltpu.sync_copy(buf, out_hbm.at[pl.ds(base + off, CHUNK)])

        gather(0, rows_a, sem_a)

        def body(p, carry):
            c0 = p * 2
            gather(c0 + 1, rows_b, sem_b)
            pltpu.make_async_copy(table_hbm.at[pl.ds(0, CHUNK)], rows_a,
                                  sem_a).wait()
            store(c0, rows_a)

            @pl.when(p + 1 < N_PAIRS)
            def _():
                gather(c0 + 2, rows_a, sem_a)

            pltpu.make_async_copy(table_hbm.at[pl.ds(0, CHUNK)], rows_b,
                                  sem_b).wait()
            store(c0 + 1, rows_b)
            return carry

        lax.fori_loop(0, N_PAIRS, body, 0, unroll=False)

    return _sc_gather


def _tc_dense_body(dense_ref, embed_ref, wb0, bb0, wb1, bb1, wb2, bb2,
                   w0a, sym, bt0, wt1, bt1, wt2, bt2, wt3, bt3, wt4, bt4,
                   out_ref):
    f32 = jnp.float32
    h = dense_ref[...]
    h = jnp.maximum(jnp.dot(h, wb0[...], preferred_element_type=f32) + bb0[...], 0.0)
    h = jnp.maximum(jnp.dot(h, wb1[...], preferred_element_type=f32) + bb1[...], 0.0)
    bot = jnp.maximum(jnp.dot(h, wb2[...], preferred_element_type=f32) + bb2[...], 0.0)

    emb = embed_ref[...].reshape(BB, NFEAT, EMBED)
    feat = lax.broadcasted_iota(jnp.int32, (BB, NFEAT, EMBED), 1)
    fs = jnp.where(feat < NSPARSE, emb,
                   jnp.where(feat == BOT_ROW, bot.reshape(BB, 1, EMBED), 0.0))
    xact = lax.dot_general(fs, fs, (((2,), (2,)), ((0,), (0,))),
                           preferred_element_type=f32)
    xflat = xact.reshape(BB, NPAIR)

    h = (jnp.dot(bot, w0a[...], preferred_element_type=f32)
         + jnp.dot(xflat, sym[...], preferred_element_type=f32) + bt0[...])
    h = jnp.maximum(h, 0.0)
    h = jnp.maximum(jnp.dot(h, wt1[...], preferred_element_type=f32) + bt1[...], 0.0)
    h = jnp.maximum(jnp.dot(h, wt2[...], preferred_element_type=f32) + bt2[...], 0.0)
    h = jnp.maximum(jnp.dot(h, wt3[...], preferred_element_type=f32) + bt3[...], 0.0)
    out_ref[...] = jnp.dot(h, wt4[...], preferred_element_type=f32) + bt4[...]


def _full_spec(shape):
    return pl.BlockSpec(shape, lambda i: (0,) * len(shape))


@functools.cache
def _make_tc_dense():
    in_specs = [
        pl.BlockSpec((BB, NDENSE), lambda i: (i, 0)),
        pl.BlockSpec((BB * NFEAT, EMBED), lambda i: (i, 0)),
        _full_spec((NDENSE, 512)), _full_spec((1, 512)),
        _full_spec((512, 256)), _full_spec((1, 256)),
        _full_spec((256, 128)), _full_spec((1, 128)),
        _full_spec((EMBED, 1024)),
        _full_spec((NPAIR, 1024)),
        _full_spec((1, 1024)),
        _full_spec((1024, 1024)), _full_spec((1, 1024)),
        _full_spec((1024, 512)), _full_spec((1, 512)),
        _full_spec((512, 256)), _full_spec((1, 256)),
        _full_spec((256, 1)), _full_spec((1, 1)),
    ]
    return pl.pallas_call(
        _tc_dense_body,
        grid=(GRID,),
        in_specs=in_specs,
        out_specs=pl.BlockSpec((BB, 1), lambda i: (i, 0)),
        out_shape=jax.ShapeDtypeStruct((SBATCH, 1), jnp.float32),
        compiler_params=pltpu.CompilerParams(
            dimension_semantics=("arbitrary",),
        ),
    )


def kernel(x, train, W_b0, b_b0, W_b1, b_b1, W_b2, b_b2, embedding_table,
           W_t0, b_t0, W_t1, b_t1, W_t2, b_t2, W_t3, b_t3, W_t4, b_t4):
    dense_in, cat_features = jnp.split(x, [NDENSE], 1)
    idx26 = jnp.asarray(cat_features, jnp.int32) % VOCAB
    idx = jnp.pad(idx26, ((0, 0), (0, NFEAT - NSPARSE))).reshape(-1)

    nf_ref = 1 + NSPARSE
    iu, ju = np.triu_indices(nf_ref)
    W0a = W_t0[:EMBED]
    W0b = W_t0[EMBED:EMBED + len(iu)]
    P = jnp.zeros((nf_ref, nf_ref, W_t0.shape[1]), W_t0.dtype)
    P = P.at[iu, ju].set(W0b)
    M = (P + P.transpose(1, 0, 2)) * 0.5
    perm = np.concatenate([np.arange(1, nf_ref), [0]])
    M = M[perm][:, perm]
    SYM = jnp.zeros((NFEAT, NFEAT, W_t0.shape[1]), W_t0.dtype)
    SYM = SYM.at[:nf_ref, :nf_ref].set(M).reshape(NPAIR, W_t0.shape[1])

    sc_gather = _make_sc_gather()
    tc_dense = _make_tc_dense()
    weights = (
        W_b0, b_b0.reshape(1, -1), W_b1, b_b1.reshape(1, -1),
        W_b2, b_b2.reshape(1, -1),
        W0a, SYM, b_t0.reshape(1, -1),
        W_t1, b_t1.reshape(1, -1), W_t2, b_t2.reshape(1, -1),
        W_t3, b_t3.reshape(1, -1), W_t4, b_t4.reshape(1, -1),
    )

    outs = []
    for s in range(NSLICE):
        embed_s = sc_gather(embedding_table,
                            lax.dynamic_slice_in_dim(idx, s * B_SLICE, B_SLICE))
        dense_s = lax.dynamic_slice_in_dim(dense_in, s * SBATCH, SBATCH)
        outs.append(tc_dense(dense_s, embed_s, *weights))
    return jnp.concatenate(outs, axis=0)


# speedup vs baseline: 10.9746x; 10.9746x over previous
import functools

import jax
import jax.numpy as jnp
import numpy as np
from jax import lax
from jax.experimental import pallas as pl
from jax.experimental.pallas import tpu as pltpu
from jax.experimental.pallas import tpu_sc as plsc

VOCAB = 1000000
EMBED = 128
NDENSE = 13
NSPARSE = 26
BATCH = 16384
NFEAT = 32
BOT_ROW = 26
NPAIR = NFEAT * NFEAT

NC = 2
NS = 16
NW = NC * NS

NSLICE = 2
SBATCH = BATCH // NSLICE
B_SLICE = SBATCH * NFEAT
B_PER_W = B_SLICE // NW
CHUNK = 256
N_CHUNKS = B_PER_W // CHUNK
N_PAIRS = N_CHUNKS // 2

BB = 512
GRID = SBATCH // BB


@functools.cache
def _make_sc_gather():
    @functools.partial(
        pl.kernel,
        mesh=plsc.VectorSubcoreMesh(core_axis_name="c", subcore_axis_name="s"),
        out_type=jax.ShapeDtypeStruct((SBATCH * NFEAT, EMBED), jnp.float32),
        scratch_types=[
            pltpu.VMEM((B_PER_W,), jnp.int32),
            pltpu.VMEM((CHUNK, EMBED), jnp.float32),
            pltpu.VMEM((CHUNK, EMBED), jnp.float32),
            pltpu.SemaphoreType.DMA,
            pltpu.SemaphoreType.DMA,
        ],
    )
    def _sc_gather(table_hbm, idx_hbm, out_hbm, idx_v, rows_a, rows_b,
                   sem_a, sem_b):
        wid = lax.axis_index("s") * NC + lax.axis_index("c")
        base = wid * B_PER_W
        pltpu.sync_copy(idx_hbm.at[pl.ds(base, B_PER_W)], idx_v)

        def gather(c, buf, sem):
            off = pl.multiple_of(c * CHUNK, CHUNK)
            return pltpu.async_copy(
                table_hbm.at[idx_v.at[pl.ds(off, CHUNK)]], buf, sem)

        def store(c, buf):
            off = pl.multiple_of(c * CHUNK, CHUNK)
            pltpu.sync_copy(buf, out_hbm.at[pl.ds(base + off, CHUNK)])

        gather(0, rows_a, sem_a)

        def body(p, carry):
            c0 = p * 2
            gather(c0 + 1, rows_b, sem_b)
            pltpu.make_async_copy(table_hbm.at[pl.ds(0, CHUNK)], rows_a,
                                  sem_a).wait()
            store(c0, rows_a)

            @pl.when(p + 1 < N_PAIRS)
            def _():
                gather(c0 + 2, rows_a, sem_a)

            pltpu.make_async_copy(table_hbm.at[pl.ds(0, CHUNK)], rows_b,
                                  sem_b).wait()
            store(c0 + 1, rows_b)
            return carry

        lax.fori_loop(0, N_PAIRS, body, 0, unroll=False)

    return _sc_gather


def _tc_dense_body(dense_ref, embed_ref, wb0, bb0, wb1, bb1, wb2, bb2,
                   w0a, sym, bt0, wt1, bt1, wt2, bt2, wt3, bt3, wt4, bt4,
                   out_ref):
    f32 = jnp.float32
    h = dense_ref[...]
    h = jnp.maximum(jnp.dot(h, wb0[...], preferred_element_type=f32) + bb0[...], 0.0)
    h = jnp.maximum(jnp.dot(h, wb1[...], preferred_element_type=f32) + bb1[...], 0.0)
    bot = jnp.maximum(jnp.dot(h, wb2[...], preferred_element_type=f32) + bb2[...], 0.0)

    emb = embed_ref[...].reshape(BB, NFEAT, EMBED)
    feat = lax.broadcasted_iota(jnp.int32, (BB, NFEAT, EMBED), 1)
    fs = jnp.where(feat < NSPARSE, emb,
                   jnp.where(feat == BOT_ROW, bot.reshape(BB, 1, EMBED), 0.0))
    xact = lax.dot_general(fs, fs, (((2,), (2,)), ((0,), (0,))),
                           preferred_element_type=f32)
    xflat = xact.reshape(BB, NPAIR)

    h = (jnp.dot(bot, w0a[...], preferred_element_type=f32)
         + jnp.dot(xflat, sym[...], preferred_element_type=f32) + bt0[...])
    h = jnp.maximum(h, 0.0)
    h = jnp.maximum(jnp.dot(h, wt1[...], preferred_element_type=f32) + bt1[...], 0.0)
    h = jnp.maximum(jnp.dot(h, wt2[...], preferred_element_type=f32) + bt2[...], 0.0)
    h = jnp.maximum(jnp.dot(h, wt3[...], preferred_element_type=f32) + bt3[...], 0.0)
    out_ref[...] = jnp.dot(h, wt4[...], preferred_element_type=f32) + bt4[...]


def _full_spec(shape):
    return pl.BlockSpec(shape, lambda i: (0,) * len(shape))


@functools.cache
def _make_tc_dense():
    in_specs = [
        pl.BlockSpec((BB, NDENSE), lambda i: (i, 0)),
        pl.BlockSpec((BB * NFEAT, EMBED), lambda i: (i, 0)),
        _full_spec((NDENSE, 512)), _full_spec((1, 512)),
        _full_spec((512, 256)), _full_spec((1, 256)),
        _full_spec((256, 128)), _full_spec((1, 128)),
        _full_spec((EMBED, 1024)),
        _full_spec((NPAIR, 1024)),
        _full_spec((1, 1024)),
        _full_spec((1024, 1024)), _full_spec((1, 1024)),
        _full_spec((1024, 512)), _full_spec((1, 512)),
        _full_spec((512, 256)), _full_spec((1, 256)),
        _full_spec((256, 1)), _full_spec((1, 1)),
    ]
    return pl.pallas_call(
        _tc_dense_body,
        grid=(GRID,),
        in_specs=in_specs,
        out_specs=pl.BlockSpec((BB, 1), lambda i: (i, 0)),
        out_shape=jax.ShapeDtypeStruct((SBATCH, 1), jnp.float32),
        compiler_params=pltpu.CompilerParams(
            dimension_semantics=("arbitrary",),
        ),
    )


def kernel(x, train, W_b0, b_b0, W_b1, b_b1, W_b2, b_b2, embedding_table,
           W_t0, b_t0, W_t1, b_t1, W_t2, b_t2, W_t3, b_t3, W_t4, b_t4):
    dense_in, cat_features = jnp.split(x, [NDENSE], 1)
    idx26 = jnp.asarray(cat_features, jnp.int32) % VOCAB
    idx = jnp.concatenate(
        [idx26, idx26[:, :NFEAT - NSPARSE]], axis=1).reshape(-1)

    nf_ref = 1 + NSPARSE
    iu, ju = np.triu_indices(nf_ref)
    W0a = W_t0[:EMBED]
    W0b = W_t0[EMBED:EMBED + len(iu)]
    P = jnp.zeros((nf_ref, nf_ref, W_t0.shape[1]), W_t0.dtype)
    P = P.at[iu, ju].set(W0b)
    M = (P + P.transpose(1, 0, 2)) * 0.5
    perm = np.concatenate([np.arange(1, nf_ref), [0]])
    M = M[perm][:, perm]
    SYM = jnp.zeros((NFEAT, NFEAT, W_t0.shape[1]), W_t0.dtype)
    SYM = SYM.at[:nf_ref, :nf_ref].set(M).reshape(NPAIR, W_t0.shape[1])

    sc_gather = _make_sc_gather()
    tc_dense = _make_tc_dense()
    weights = (
        W_b0, b_b0.reshape(1, -1), W_b1, b_b1.reshape(1, -1),
        W_b2, b_b2.reshape(1, -1),
        W0a, SYM, b_t0.reshape(1, -1),
        W_t1, b_t1.reshape(1, -1), W_t2, b_t2.reshape(1, -1),
        W_t3, b_t3.reshape(1, -1), W_t4, b_t4.reshape(1, -1),
    )

    outs = []
    for s in range(NSLICE):
        embed_s = sc_gather(embedding_table,
                            lax.dynamic_slice_in_dim(idx, s * B_SLICE, B_SLICE))
        dense_s = lax.dynamic_slice_in_dim(dense_in, s * SBATCH, SBATCH)
        outs.append(tc_dense(dense_s, embed_s, *weights))
    return jnp.concatenate(outs, axis=0)
